# baseline (speedup 1.0000x reference)
import functools

import jax
import jax.numpy as jnp
from jax import lax
from jax.experimental import pallas as pl
from jax.experimental.pallas import tpu as pltpu
from jax.experimental.pallas import tpu_sc as plsc

D = 32
BM = 1024
NW = 32
IDX_CHUNK = 128


def _argmin_body(x_ref, w_ref, sw_ref, iota_ref, idx_ref, loss_ref):
    i = pl.program_id(0)
    k = w_ref.shape[0]
    xb = x_ref[...]
    m = lax.dot_general(
        xb, w_ref[...], (((1,), (1,)), ((), ())),
        preferred_element_type=jnp.float32)
    sx = jnp.sum(xb * xb, axis=1, keepdims=True)
    d = (sx + sw_ref[...]) - 2.0 * m
    dmin = jnp.min(d, axis=1, keepdims=True)
    cand = jnp.where(d == dmin, iota_ref[...], jnp.float32(k))
    idx_ref[0, 0, :] = jnp.min(cand, axis=1).astype(jnp.int32)

    @pl.when(i == 0)
    def _():
        loss_ref[...] = jnp.zeros_like(loss_ref)

    loss_ref[...] += jnp.sum(dmin).reshape(1, 1)


def _argmin_call(flat_x, W, sw, iota_f):
    n = flat_x.shape[0]
    k = W.shape[0]
    nb = n // BM
    return pl.pallas_call(
        _argmin_body,
        grid=(nb,),
        in_specs=[
            pl.BlockSpec((BM, D), lambda i: (i, 0)),
            pl.BlockSpec((k, D), lambda i: (0, 0)),
            pl.BlockSpec((1, k), lambda i: (0, 0)),
            pl.BlockSpec((1, k), lambda i: (0, 0)),
        ],
        out_specs=[
            pl.BlockSpec((1, 1, BM), lambda i: (i, 0, 0)),
            pl.BlockSpec((1, 1), lambda i: (0, 0)),
        ],
        out_shape=[
            jax.ShapeDtypeStruct((nb, 1, BM), jnp.int32),
            jax.ShapeDtypeStruct((1, 1), jnp.float32),
        ],
    )(flat_x, W, sw, iota_f)


def _sc_gather(W, idx3):
    ch = idx3.shape[1]
    bpw = ch * IDX_CHUNK
    mesh = plsc.VectorSubcoreMesh(core_axis_name="c", subcore_axis_name="s")

    @functools.partial(
        pl.kernel,
        out_type=jax.ShapeDtypeStruct((NW, bpw, D), jnp.float32),
        mesh=mesh,
        compiler_params=pltpu.CompilerParams(use_tc_tiling_on_sc=False),
        scratch_types=[
            pltpu.VMEM((ch, IDX_CHUNK), jnp.int32),
            pltpu.VMEM((bpw, D), jnp.float32),
            pltpu.SemaphoreType.DMA,
        ],
    )
    def body(w_hbm, idx_hbm, out_hbm, idx_v, rows_v, sem):
        wid = lax.axis_index("s") * 2 + lax.axis_index("c")
        pltpu.sync_copy(idx_hbm.at[wid], idx_v)
        copies = [
            pltpu.async_copy(
                w_hbm.at[idx_v.at[j]],
                rows_v.at[pl.ds(j * IDX_CHUNK, IDX_CHUNK)],
                sem,
            )
            for j in range(ch)
        ]
        for cp in copies:
            cp.wait()
        pltpu.sync_copy(rows_v, out_hbm.at[wid])

    return body(W, idx3)


def kernel(x, W):
    n = x.shape[0] * x.shape[1]
    k = W.shape[0]
    flat_x = x.reshape(n, D)
    sw = jnp.sum(W ** 2, axis=1).reshape(1, k)
    iota_f = lax.iota(jnp.float32, k).reshape(1, k)

    bpw = n // NW
    idx3, loss = _argmin_call(flat_x, W, sw, iota_f)
    idx_w = idx3.reshape(NW, bpw // IDX_CHUNK, IDX_CHUNK)
    out = _sc_gather(W, idx_w)
    quantized_st = out.reshape(x.shape)
    per_elem = loss[0, 0] / jnp.float32(n * D)
    vq_loss = per_elem + 0.25 * per_elem
    return quantized_st, vq_loss


# speedup vs baseline: 1.0422x; 1.0422x over previous
import functools

import jax
import jax.numpy as jnp
from jax import lax
from jax.experimental import pallas as pl
from jax.experimental.pallas import tpu as pltpu
from jax.experimental.pallas import tpu_sc as plsc

D = 32
BM = 1024
NW = 32
IDX_CHUNK = 128


def _argmin_body(x_ref, w_ref, sw_ref, iota_ref, idx_ref, loss_ref):
    i = pl.program_id(0)
    k = w_ref.shape[0]
    xb = x_ref[...]
    m = lax.dot_general(
        xb, w_ref[...], (((1,), (1,)), ((), ())),
        preferred_element_type=jnp.float32)
    sx = jnp.sum(xb * xb, axis=1, keepdims=True)
    d = (sx + sw_ref[...]) - 2.0 * m
    dmin = jnp.min(d, axis=1, keepdims=True)
    cand = jnp.where(d == dmin, iota_ref[...], jnp.float32(k))
    idx = jnp.min(cand, axis=1).astype(jnp.int32)
    idx_ref[0, :, :] = idx.reshape(BM // 128, 128)

    @pl.when(i == 0)
    def _():
        loss_ref[...] = jnp.zeros_like(loss_ref)

    loss_ref[...] += jnp.sum(dmin).reshape(1, 1)


def _argmin_call(flat_x, W, sw, iota_f):
    n = flat_x.shape[0]
    k = W.shape[0]
    nb = n // BM
    return pl.pallas_call(
        _argmin_body,
        grid=(nb,),
        in_specs=[
            pl.BlockSpec((BM, D), lambda i: (i, 0)),
            pl.BlockSpec((k, D), lambda i: (0, 0)),
            pl.BlockSpec((1, k), lambda i: (0, 0)),
            pl.BlockSpec((1, k), lambda i: (0, 0)),
        ],
        out_specs=[
            pl.BlockSpec((1, BM // 128, 128), lambda i: (i, 0, 0)),
            pl.BlockSpec((1, 1), lambda i: (0, 0)),
        ],
        out_shape=[
            jax.ShapeDtypeStruct((nb, BM // 128, 128), jnp.int32),
            jax.ShapeDtypeStruct((1, 1), jnp.float32),
        ],
    )(flat_x, W, sw, iota_f)


def _sc_gather(W, idx2):
    rows_per_w = idx2.shape[0] // NW
    ch = rows_per_w
    bpw = ch * IDX_CHUNK
    mesh = plsc.VectorSubcoreMesh(core_axis_name="c", subcore_axis_name="s")

    @functools.partial(
        pl.kernel,
        out_type=jax.ShapeDtypeStruct((NW, bpw, D), jnp.float32),
        mesh=mesh,
        compiler_params=pltpu.CompilerParams(use_tc_tiling_on_sc=False),
        scratch_types=[
            pltpu.VMEM((ch, IDX_CHUNK), jnp.int32),
            pltpu.VMEM((bpw, D), jnp.float32),
            pltpu.SemaphoreType.DMA,
        ],
    )
    def body(w_hbm, idx_hbm, out_hbm, idx_v, rows_v, sem):
        wid = lax.axis_index("s") * 2 + lax.axis_index("c")
        pltpu.sync_copy(idx_hbm.at[pl.ds(wid * rows_per_w, rows_per_w)], idx_v)
        copies = [
            pltpu.async_copy(
                w_hbm.at[idx_v.at[j]],
                rows_v.at[pl.ds(j * IDX_CHUNK, IDX_CHUNK)],
                sem,
            )
            for j in range(ch)
        ]
        for cp in copies:
            cp.wait()
        pltpu.sync_copy(rows_v, out_hbm.at[wid])

    return body(W, idx2)


def kernel(x, W):
    n = x.shape[0] * x.shape[1]
    k = W.shape[0]
    flat_x = x.reshape(n, D)
    sw = jnp.sum(W ** 2, axis=1).reshape(1, k)
    iota_f = lax.iota(jnp.float32, k).reshape(1, k)

    idx3, loss = _argmin_call(flat_x, W, sw, iota_f)
    out = _sc_gather(W, idx3.reshape(n // IDX_CHUNK, IDX_CHUNK))
    quantized_st = out.reshape(x.shape)
    per_elem = loss[0, 0] / jnp.float32(n * D)
    vq_loss = per_elem + 0.25 * per_elem
    return quantized_st, vq_loss
